# baseline (speedup 1.0000x reference)
import functools

import jax
import jax.numpy as jnp
from jax import lax
from jax.experimental import pallas as pl
from jax.experimental.pallas import tpu as pltpu
from jax.experimental.pallas import tpu_sc as plsc

_VOCAB = 100000
_D = 4 * 4 * 16
_NI = 4096
_NJ = 26
_NW = 32
_L = 128

_mesh = plsc.VectorSubcoreMesh(core_axis_name="c", subcore_axis_name="s")


@functools.partial(
    pl.kernel,
    mesh=_mesh,
    out_type=jax.ShapeDtypeStruct((_NJ, 4, 4, 2, _NW, 8, _L), jnp.float32),
    compiler_params=pltpu.CompilerParams(needs_layout_passes=False),
    scratch_types=[
        pltpu.VMEM((_NJ, _L), jnp.int32),
        pltpu.VMEM((_L, _D), jnp.float32),
        pltpu.VMEM((_L, _D), jnp.float32),
        pltpu.VMEM((_D // 2, _L), jnp.float32),
        pltpu.VMEM((_D // 2, _L), jnp.float32),
        pltpu.SemaphoreType.DMA,
        pltpu.SemaphoreType.DMA,
        pltpu.SemaphoreType.DMA,
        pltpu.SemaphoreType.DMA,
    ],
)
def _sc_gather(idx_hbm, table_hbm, out_hbm, idx_v, rows0, rows1, rt0, rt1,
               g0, g1, w0, w1):
    cid = lax.axis_index("c")
    sid = lax.axis_index("s")
    wid = sid * 2 + cid
    iota = lax.iota(jnp.int32, 16)

    pltpu.sync_copy(idx_hbm.at[:, wid], idx_v)

    def gather(j, rows, sem):
        pltpu.async_copy(table_hbm.at[idx_v.at[j]], rows, sem)

    def wait_gather(rows, sem):
        pltpu.make_async_copy(table_hbm.at[idx_v.at[0]], rows, sem).wait()

    perms = [(iota + k) & 15 for k in range(16)]

    def transpose_half(rows, rt, p0):
        @plsc.parallel_loop(0, 64, unroll=2)
        def tbody(blk):
            pb = blk >> 3
            l0 = blk & 7
            lrow = iota + l0 * 16
            for k in range(16):
                prel = perms[k] + pb * 16
                vals = plsc.load_gather(rows, [lrow, prel + p0])
                plsc.store_scatter(rt, [prel, lrow], vals)

    def emit_half(j, rt, half, sem):
        for slab in range(16):
            q = half * 16 + slab
            a, b, g = q >> 3, (q >> 1) & 3, q & 1
            pltpu.async_copy(rt.at[pl.ds(slab * 8, 8)],
                             out_hbm.at[j, a, b, g, wid], sem)

    def drain_half(rt, sem):
        for _ in range(16):
            pltpu.make_async_copy(rt.at[pl.ds(0, 8)],
                                  out_hbm.at[0, 0, 0, 0, wid], sem).wait()

    def item(j, rows, gsem):
        wait_gather(rows, gsem)

        @pl.when(j > 0)
        def _():
            drain_half(rt0, w0)

        transpose_half(rows, rt0, 0)
        emit_half(j, rt0, 0, w0)

        @pl.when(j > 0)
        def _():
            drain_half(rt1, w1)

        transpose_half(rows, rt1, 128)
        emit_half(j, rt1, 1, w1)

    gather(0, rows0, g0)

    def body(j2, carry):
        e = 2 * j2
        gather(e + 1, rows1, g1)
        item(e, rows0, g0)

        @pl.when(e + 2 < _NJ)
        def _():
            gather(e + 2, rows0, g0)

        item(e + 1, rows1, g1)
        return carry

    lax.fori_loop(0, _NJ // 2, body, 0)
    drain_half(rt0, w0)
    drain_half(rt1, w1)


def kernel(inputs, kernel):
    table = kernel.reshape(_VOCAB, _D)
    idx = inputs.T.reshape(_NJ, _NW, _L)
    x7 = _sc_gather(idx, table)
    return x7.transpose(4, 6, 0, 1, 2, 3, 5).reshape(_NI, _NJ, 4, 4, 16)
